# Initial kernel scaffold; baseline (speedup 1.0000x reference)
import jax
import jax.numpy as jnp
from jax.experimental import pallas as pl


def kernel(idi, dismatrix_eu, locations):
    raise NotImplementedError("write your pallas kernel here")


# speedup vs baseline: 1181.4856x; 1181.4856x over previous
import contextlib
import functools

import numpy as np
import jax
import jax.numpy as jnp
from jax import lax
from jax.experimental import pallas as pl
from jax.experimental.pallas import tpu as pltpu
from jax.experimental.pallas import tpu_sc as plsc

_H = 16
_TP = 8
_NB = 4
_BROWS = 16
_M = 129
_LANES = _M * _TP
_P = 2048
_LUT_FIX = (32767,)


def _gen_consts():
    try:
        ctx = jax.default_device(jax.devices("cpu")[0])
    except Exception:
        ctx = contextlib.nullcontext()
    with ctx:
        key = jax.random.key(42)
        key, ks = jax.random.split(key)
        perm = np.asarray(jax.random.permutation(ks, 64))
        xa_list, sgn_list, maskt_list = [], [], []
        flip = (1 << np.arange(_H)).astype(np.int64)
        for _b in range(_NB):
            key, kb = jax.random.split(key)
            k1, k2, k3 = jax.random.split(kb, 3)
            k1a, k1b = jax.random.split(k1)
            probs = np.asarray(jax.random.uniform(k1a, (_BROWS, 32)))
            choosing = probs > 0.2
            cols = np.asarray(jax.random.randint(k1b, (_BROWS, 32), 0, 32))
            onehot = cols[:, :, None] == np.arange(32)[None, None, :]
            mask = np.where(~choosing[:, :, None], onehot, True)
            upper = (2 ** jnp.arange(_H, dtype=jnp.int32)).astype(jnp.int32)
            rn = jax.random.randint(k2, (_H, 512, 4, _TP), 0,
                                    1 << _H, dtype=jnp.int32)
            rm = np.asarray(jnp.transpose(rn % upper[:, None, None, None],
                                          (1, 0, 2, 3)))
            p129 = np.asarray(jax.random.permutation(k3, _M))
            xa = np.zeros((512, _M, _TP), np.int32)
            sgn = np.zeros((_M,), np.float32)
            for mslot in range(_M):
                o = int(p129[mslot])
                if o == 64:
                    sgn[mslot] = 1.0
                else:
                    if o < 64:
                        h, kk = o // 4, o % 4
                        sgn[mslot] = 1.0
                    else:
                        o2 = o - 65
                        h, kk = o2 // 4, o2 % 4
                        sgn[mslot] = -1.0
                    xa[:, mslot, :] = np.int32(flip[h]) ^ rm[:, h, kk, :]
            xa_list.append(xa.reshape(512, _LANES))
            sgn_list.append(np.repeat(sgn, _TP))
            maskt_list.append(mask.astype(np.float32))
        xa_all = np.stack(xa_list).astype(np.int32)
        sgn_all = np.stack(sgn_list)[:, None, :]
        maskt_all = np.concatenate(maskt_list)
        inv = np.argsort(perm)
        return perm, inv, xa_all, sgn_all, maskt_all


_PERM, _INVPERM, _XA, _SGN, _MASKT = _gen_consts()


def _main_body(rows16, ids_col, ids_row, xar, sgnf, eu3, maskt,
               out, t_ref, sel_ref):
    f32 = jnp.float32
    i32 = jnp.int32

    def _sgn(v):
        return jnp.where(v > 0, 1.0, jnp.where(v < 0, -1.0, 0.0)).astype(f32)

    def _lut(xr):
        v = (xr + 1).astype(f32)
        e = (lax.bitcast_convert_type(v, i32) >> 23) - 127
        for bad in _LUT_FIX:
            e = e - (xr == bad).astype(i32)
        return (e + 1).astype(f32) * (1.0 / 16.0)

    def _tile_lanes(a, rep):
        cur, n = a, 1
        while n * 2 <= rep:
            cur = jnp.concatenate([cur, cur], axis=-1)
            n *= 2
        while n < rep:
            cur = jnp.concatenate([cur, a], axis=-1)
            n += 1
        return cur

    sub = ids_col[...] & 15
    cur = rows16[:, 0:8]
    for k in range(1, 16):
        cur = jnp.where(sub == k, rows16[:, 8 * k:8 * k + 8], cur)
    t_ref[...] = cur

    for b in range(_NB):
        sgl = sgnf[b]

        def row_body(r, carry, b=b, sgl=sgl):
            base = b * 512 + r * 32
            x = t_ref[pl.ds(base, 32), :]
            ax = jnp.abs(x)
            sx = _sgn(x)
            ax_l = _tile_lanes(ax, _M)
            sx_l = _tile_lanes(sx, _M)
            xa = xar[b, pl.ds(r * 32, 32), :]
            v_cand = ax_l ^ xa
            nzf = (v_cand > 0).astype(f32)
            eu_st = eu3[b * 16 + r]
            mk_st = maskt[b * 16 + r]

            acc = None
            for s in range(32):
                ax_row = ax_l[s:s + 1, :]
                sx_row = sx_l[s:s + 1, :]
                d_s = sx_row * sgl * nzf * (1.0 - _lut(ax_row ^ v_cand))
                d2_s = sx * sx[s:s + 1, :] * (1.0 - _lut(ax ^ ax[s:s + 1, :]))
                q_s = jnp.sum(d2_s, axis=1, keepdims=True)
                t1 = (d_s - _tile_lanes(d2_s, _M) + q_s) * 0.125
                term = jnp.abs(t1 - eu_st[:, s:s + 1]) * mk_st[:, s:s + 1]
                acc = term if acc is None else acc + term

            idx0 = lax.broadcasted_iota(i32, (32, _LANES), 1) >> 3
            cval = sgl * v_cand.astype(f32)

            def lexmin(la, ia, ca, lb, ib, cb):
                ta = (la < lb) | ((la == lb) & (ia <= ib))
                return (jnp.where(ta, la, lb), jnp.where(ta, ia, ib),
                        jnp.where(ta, ca, cb))

            lc, ic, cc = lexmin(acc[:, :512], idx0[:, :512], cval[:, :512],
                                acc[:, 512:1024], idx0[:, 512:1024],
                                cval[:, 512:1024])
            w = 256
            while w >= 8:
                lc, ic, cc = lexmin(lc[:, :w], ic[:, :w], cc[:, :w],
                                    lc[:, w:2 * w], ic[:, w:2 * w],
                                    cc[:, w:2 * w])
                w //= 2
            lc, ic, cc = lexmin(lc, ic, cc, acc[:, 1024:1032],
                                idx0[:, 1024:1032], cval[:, 1024:1032])
            sel_ref[pl.ds(r * 32, 32), :] = cc.astype(i32)
            return carry

        lax.fori_loop(0, _BROWS, row_body, None)

        bids = ids_row[0:1, b * 512:(b + 1) * 512]
        eq = ids_col[...] == bids
        iu = lax.broadcasted_iota(i32, (_P, 512), 1)
        umax = jnp.max(jnp.where(eq, iu, -1), axis=1, keepdims=True)
        oh = (iu == umax).astype(f32)
        sel = sel_ref[...]
        lo = jnp.dot(oh, (sel & 255).astype(f32), preferred_element_type=f32)
        hi = jnp.dot(oh, (sel >> 8).astype(f32), preferred_element_type=f32)
        upd = hi.astype(i32) * 256 + lo.astype(i32)
        t_ref[...] = jnp.where(umax >= 0, upd, t_ref[...])

    t = t_ref[...]
    t4 = t.reshape(64, 32, 8)
    at = jnp.abs(t)
    st = _sgn(t)
    cols = []
    for j in range(32):
        bj = jnp.broadcast_to(t4[:, j:j + 1, :], (64, 32, 8)).reshape(_P, 8)
        d3 = st * _sgn(bj) * (1.0 - _lut(at ^ jnp.abs(bj)))
        cols.append(jnp.sum(d3, axis=1, keepdims=True) * (1.0 / _TP))
    out[...] = jnp.concatenate(cols, axis=1)


_SC_GATHER = None


def _build_sc_gather():
    info = plsc.get_sparse_core_info()
    nc, ns = info.num_cores, info.num_subcores
    nw = nc * ns
    bpw = _P // nw
    mesh = plsc.VectorSubcoreMesh(core_axis_name="c", subcore_axis_name="s")

    @functools.partial(
        pl.kernel, mesh=mesh,
        out_type=jax.ShapeDtypeStruct((_P, 128), jnp.int32),
        scratch_types=[
            pltpu.VMEM((bpw,), jnp.int32),
            pltpu.VMEM((bpw, 128), jnp.int32),
            pltpu.SemaphoreType.DMA,
        ],
    )
    def sc_gather(table_hbm, idx_hbm, out_hbm, idx_v, rows_v, sem):
        wid = lax.axis_index("s") * nc + lax.axis_index("c")
        base = wid * bpw
        pltpu.sync_copy(idx_hbm.at[pl.ds(base, bpw)], idx_v)
        pltpu.async_copy(table_hbm.at[idx_v], rows_v, sem).wait()
        pltpu.sync_copy(rows_v, out_hbm.at[pl.ds(base, bpw)])

    return sc_gather


def kernel(idi, dismatrix_eu, locations):
    global _SC_GATHER
    ids_all = idi[_PERM].reshape(-1).astype(jnp.int32)
    eu3 = dismatrix_eu[_PERM].astype(jnp.float32)

    if _SC_GATHER is None:
        _SC_GATHER = _build_sc_gather()
    table = locations.reshape(4096, 128)
    rows16 = _SC_GATHER(table, ids_all >> 4)

    out2 = pl.pallas_call(
        _main_body,
        out_shape=jax.ShapeDtypeStruct((_P, 32), jnp.float32),
        scratch_shapes=[
            pltpu.VMEM((_P, 8), jnp.int32),
            pltpu.VMEM((512, 8), jnp.int32),
        ],
    )(rows16, ids_all.reshape(_P, 1),
      jnp.broadcast_to(ids_all[None, :], (8, _P)),
      jnp.asarray(_XA), jnp.asarray(_SGN), eu3, jnp.asarray(_MASKT))
    return out2.reshape(64, 32, 32)[_INVPERM]
